# baseline (speedup 1.0000x reference)
import numpy as np
import jax
import jax.numpy as jnp
from jax.experimental import pallas as pl
from jax.experimental.pallas import tpu as pltpu

B = 16384
A = 6
SLOTS = 6
GREY = 5.0
YELLOW = 4.0
FEATURE_DIM = 256
FW = 16
XW = (1 + SLOTS) * FW
R = 2048


def _consts():
    lt = np.zeros((8, 8), np.float32)
    for i in range(A):
        for j in range(A):
            if i <= j:
                lt[i, j] = 1.0
    e8 = np.zeros((8, 96), np.float32)
    for a in range(A):
        e8[a, 16 * a:16 * a + 16] = 1.0
    et = np.zeros((96, FW), np.float32)
    for a in range(A):
        for j in range(FW):
            et[16 * a + j, j] = 1.0
    return jnp.asarray(lt), jnp.asarray(e8), jnp.asarray(et)


def _fused(cf_ref, f_ref, qp_ref, w_ref, lt_ref, e8_ref, et_ref,
           o_ref, x_ref):
    cf = cf_ref[...]
    one = jnp.float32(1.0)
    zero = jnp.float32(0.0)
    mg = jnp.where(cf == GREY, one, zero)
    my = jnp.where(cf == YELLOW, one, zero)
    ug = mg * jnp.dot(mg, lt_ref[...], preferred_element_type=jnp.float32)
    uy = my * jnp.dot(my, lt_ref[...], preferred_element_type=jnp.float32)

    f = f_ref[...]
    x_ref[:, 0:FW] = qp_ref[...]
    for p in range(SLOTS):
        u = ug if p < 2 else uy
        tgt = jnp.float32(p + 1 if p < 2 else p - 1)
        s = jnp.where(u == tgt, one, zero)
        mp = jnp.dot(s, e8_ref[...],
                     preferred_element_type=jnp.float32)
        xs = jnp.dot(mp * f, et_ref[...],
                     preferred_element_type=jnp.float32)
        x_ref[:, FW * (1 + p):FW * (2 + p)] = xs
    acc = jnp.dot(x_ref[...], w_ref[...], preferred_element_type=jnp.float32)
    o_ref[...] = jnp.maximum(acc, 0.0)


def kernel(query_position, query_direction, query_abilities, query_carried,
           query_status, all_agent_positions, all_agent_directions,
           all_agent_abilities, all_agent_carried, all_agent_status,
           agent_color_indices, W, b):
    feats = jnp.concatenate([all_agent_positions, all_agent_directions,
                             all_agent_abilities, all_agent_carried,
                             all_agent_status], axis=-1)
    F = jnp.pad(feats, ((0, 0), (0, 0), (0, FW - 13))).reshape(B, A * FW)

    cf = jnp.pad(agent_color_indices.astype(jnp.float32),
                 ((0, 0), (0, 8 - A)), constant_values=-1.0)

    q = jnp.concatenate([query_position, query_direction, query_abilities,
                         query_carried, query_status], axis=1)
    qp = jnp.concatenate([q, jnp.ones((B, 1), q.dtype),
                          jnp.zeros((B, FW - 14), q.dtype)], axis=1)

    wq = jnp.concatenate([W[:13], b[None, :],
                          jnp.zeros((FW - 14, FEATURE_DIM), W.dtype)])
    ws = jnp.pad(W[13:].reshape(SLOTS, 13, FEATURE_DIM),
                 ((0, 0), (0, FW - 13), (0, 0))).reshape(SLOTS * FW,
                                                         FEATURE_DIM)
    w112 = jnp.concatenate([wq, ws], axis=0)

    lt, e8, et = _consts()

    rep = lambda i: (0, 0)
    row = lambda i: (i, 0)
    out = pl.pallas_call(
        _fused,
        grid=(B // R,),
        in_specs=[
            pl.BlockSpec((R, 8), row),
            pl.BlockSpec((R, A * FW), row),
            pl.BlockSpec((R, FW), row),
            pl.BlockSpec((XW, FEATURE_DIM), rep),
            pl.BlockSpec((8, 8), rep),
            pl.BlockSpec((8, 96), rep),
            pl.BlockSpec((96, FW), rep),
        ],
        out_specs=pl.BlockSpec((R, FEATURE_DIM), row),
        out_shape=jax.ShapeDtypeStruct((B, FEATURE_DIM), jnp.float32),
        scratch_shapes=[pltpu.VMEM((R, XW), jnp.float32)],
    )(cf, F, qp, w112, lt, e8, et)
    return out
